# baseline (speedup 1.0000x reference)
import functools

import jax
import jax.numpy as jnp
import numpy as np
from jax import lax
from jax.experimental import pallas as pl
from jax.experimental.pallas import tpu as pltpu
from jax.experimental.pallas import tpu_sc as plsc

N_LEVELS = 16
F_PER_LEVEL = 2
T = 2 ** 19
BASE_RES = 16
PER_LEVEL_SCALE = 1.5
P1 = int(np.uint32(2654435761).view(np.int32))
P2 = int(np.uint32(805459861).view(np.int32))

NC, NS = 2, 16
NW = NC * NS
L = 16
C = 128


def _level_params():
    params = []
    for l in range(N_LEVELS):
        scale = BASE_RES * (PER_LEVEL_SCALE ** l) - 1.0
        res = int(np.ceil(scale)) + 1
        dense = (res ** 3) <= T
        params.append((float(scale), res, dense))
    return params

LEVELS = _level_params()


def _corner_bits(corner):
    return (corner >> 0) & 1, (corner >> 1) & 1, (corner >> 2) & 1


def _make_enc_kernel(n_points):
    pts_per_w = n_points // NW
    n_chunks = pts_per_w // C
    groups = C // L

    mesh = plsc.VectorSubcoreMesh(
        core_axis_name="c", subcore_axis_name="s",
        num_cores=NC, num_subcores=NS)

    @functools.partial(
        pl.kernel,
        mesh=mesh,
        compiler_params=pltpu.CompilerParams(use_tc_tiling_on_sc=False,
                                             needs_layout_passes=False),
        out_type=jax.ShapeDtypeStruct((n_points, N_LEVELS * F_PER_LEVEL),
                                      jnp.float32),
        scratch_types=[
            pltpu.VMEM((C,), jnp.float32),
            pltpu.VMEM((C,), jnp.float32),
            pltpu.VMEM((C,), jnp.float32),
            pltpu.VMEM((8, C), jnp.int32),
            pltpu.VMEM((8, C), jnp.int32),
            pltpu.VMEM((8, C), jnp.float32),
            pltpu.VMEM((8 * C, 8), jnp.float32),
            pltpu.VMEM((C, N_LEVELS * F_PER_LEVEL), jnp.float32),
            pltpu.SemaphoreType.DMA,
        ],
    )
    def enc_kernel(xa, xb, xc, table8, out, xva, xvb, xvc,
                   idxv, offv, wv, rowsv, encv, sem):
        wid = lax.axis_index("s") * NC + lax.axis_index("c")
        base0 = wid * pts_per_w
        xvs = (xva, xvb, xvc)

        def chunk_body(ch, carry):
            base = base0 + ch * C
            pltpu.sync_copy(xa.at[pl.ds(base, C)], xva)
            pltpu.sync_copy(xb.at[pl.ds(base, C)], xvb)
            pltpu.sync_copy(xc.at[pl.ds(base, C)], xvc)

            for l, (scale, res, dense) in enumerate(LEVELS):
                lvl_base = l * T

                def pass_a(g, c2, scale=scale, res=res, dense=dense,
                           lvl_base=lvl_base):
                    off = g * L
                    sl = pl.ds(off, L)
                    coords = []
                    for d in range(3):
                        x01 = (xvs[d][sl] + 1.0) * 0.5
                        pos = x01 * jnp.float32(scale) + 0.5
                        pg = pos.astype(jnp.int32)
                        fr = pos - pg.astype(jnp.float32)
                        coords.append((pg, fr))
                    (pgx, fx), (pgy, fy), (pgz, fz) = coords
                    wx = (1.0 - fx, fx)
                    wy = (1.0 - fy, fy)
                    wz = (1.0 - fz, fz)
                    wxy = {(bx, by): wx[bx] * wy[by]
                           for bx in (0, 1) for by in (0, 1)}
                    if dense:
                        r1 = jnp.int32(res - 1)
                        cx = (jnp.minimum(pgx, r1), jnp.minimum(pgx + 1, r1))
                        cy0 = jnp.minimum(pgy, r1) * jnp.int32(res)
                        cy1 = jnp.minimum(pgy + 1, r1) * jnp.int32(res)
                        cz0 = jnp.minimum(pgz, r1) * jnp.int32(res * res)
                        cz1 = jnp.minimum(pgz + 1, r1) * jnp.int32(res * res)
                        cy = (cy0, cy1)
                        cz = (cz0, cz1)
                        for corner in range(8):
                            bx, by, bz = _corner_bits(corner)
                            e = cx[bx] + cy[by] + cz[bz] + jnp.int32(lvl_base)
                            idxv[corner, sl] = lax.shift_right_logical(
                                e, jnp.int32(2))
                            offv[corner, sl] = lax.shift_left(
                                e & jnp.int32(3), jnp.int32(1))
                            wv[corner, sl] = wxy[(bx, by)] * wz[bz]
                    else:
                        hx = (pgx, pgx + 1)
                        hy0 = pgy * jnp.int32(P1)
                        hy = (hy0, hy0 + jnp.int32(P1))
                        hz0 = pgz * jnp.int32(P2)
                        hz = (hz0, hz0 + jnp.int32(P2))
                        for corner in range(8):
                            bx, by, bz = _corner_bits(corner)
                            e = ((hx[bx] ^ hy[by] ^ hz[bz]) & jnp.int32(T - 1)
                                 ) + jnp.int32(lvl_base)
                            idxv[corner, sl] = lax.shift_right_logical(
                                e, jnp.int32(2))
                            offv[corner, sl] = lax.shift_left(
                                e & jnp.int32(3), jnp.int32(1))
                            wv[corner, sl] = wxy[(bx, by)] * wz[bz]
                    return c2

                lax.fori_loop(0, groups, pass_a, 0)

                cps = [pltpu.async_copy(table8.at[idxv.at[corner]],
                                        rowsv.at[pl.ds(corner * C, C)], sem)
                       for corner in range(8)]
                for cp in cps:
                    cp.wait()

                def pass_b(g, c2, l=l):
                    off = g * L
                    sl = pl.ds(off, L)
                    pvec = lax.iota(jnp.int32, L) + off
                    acc0 = jnp.zeros((L,), jnp.float32)
                    acc1 = jnp.zeros((L,), jnp.float32)
                    for corner in range(8):
                        rvec = pvec + jnp.int32(corner * C)
                        ov = offv[corner, sl]
                        w = wv[corner, sl]
                        f0 = plsc.load_gather(rowsv, [rvec, ov])
                        f1 = plsc.load_gather(rowsv, [rvec, ov + 1])
                        acc0 = acc0 + w * f0
                        acc1 = acc1 + w * f1
                    col0 = jnp.full((L,), 2 * l, jnp.int32)
                    plsc.store_scatter(encv, [pvec, col0], acc0)
                    plsc.store_scatter(encv, [pvec, col0 + 1], acc1)
                    return c2

                lax.fori_loop(0, groups, pass_b, 0)

            pltpu.sync_copy(encv, out.at[pl.ds(base, C)])
            return carry

        lax.fori_loop(0, n_chunks, chunk_body, 0)

    return enc_kernel


def _mlp_body(enc_ref, w1_ref, b1_ref, w2_ref, b2_ref, out_ref):
    h = jnp.dot(enc_ref[...], w1_ref[...],
                preferred_element_type=jnp.float32) + b1_ref[...]
    h = jnp.maximum(h, 0.0)
    out_ref[...] = jnp.dot(h, w2_ref[...],
                           preferred_element_type=jnp.float32) + b2_ref[...]


def _mlp(enc, W1, b1, W2, b2):
    n = enc.shape[0]
    bm = 4096
    return pl.pallas_call(
        _mlp_body,
        grid=(n // bm,),
        in_specs=[
            pl.BlockSpec((bm, N_LEVELS * F_PER_LEVEL), lambda i: (i, 0)),
            pl.BlockSpec((N_LEVELS * F_PER_LEVEL, 64), lambda i: (0, 0)),
            pl.BlockSpec((1, 64), lambda i: (0, 0)),
            pl.BlockSpec((64, 16), lambda i: (0, 0)),
            pl.BlockSpec((1, 16), lambda i: (0, 0)),
        ],
        out_specs=pl.BlockSpec((bm, 16), lambda i: (i, 0)),
        out_shape=jax.ShapeDtypeStruct((n, 16), jnp.float32),
    )(enc, W1, b1.reshape(1, 64), W2, b2.reshape(1, 16))


def kernel(x, table, W1, b1, W2, b2):
    n = x.shape[0]
    xT = x.T
    table8 = table.reshape(N_LEVELS * T * F_PER_LEVEL // 8, 8)
    enc = _make_enc_kernel(n)(xT[0], xT[1], xT[2], table8)
    out = _mlp(enc, W1, b1, W2, b2)
    return out.reshape(x.shape[:-1] + (16,))


# speedup vs baseline: 3.2971x; 3.2971x over previous
import functools

import jax
import jax.numpy as jnp
import numpy as np
from jax import lax
from jax.experimental import pallas as pl
from jax.experimental.pallas import tpu as pltpu
from jax.experimental.pallas import tpu_sc as plsc

N_LEVELS = 16
F_PER_LEVEL = 2
T = 2 ** 19
BASE_RES = 16
PER_LEVEL_SCALE = 1.5
P1 = int(np.uint32(2654435761).view(np.int32))
P2 = int(np.uint32(805459861).view(np.int32))

NC, NS = 2, 16
NW = NC * NS
L = 16
C = 128


def _level_params():
    params = []
    for l in range(N_LEVELS):
        scale = BASE_RES * (PER_LEVEL_SCALE ** l) - 1.0
        res = int(np.ceil(scale)) + 1
        dense = (res ** 3) <= T
        params.append((float(scale), res, dense))
    return params

LEVELS = _level_params()


def _corner_bits(corner):
    return (corner >> 0) & 1, (corner >> 1) & 1, (corner >> 2) & 1


def _make_enc_kernel(n_points):
    pts_per_w = n_points // NW
    n_chunks = pts_per_w // C
    groups = C // L

    mesh = plsc.VectorSubcoreMesh(
        core_axis_name="c", subcore_axis_name="s",
        num_cores=NC, num_subcores=NS)

    @functools.partial(
        pl.kernel,
        mesh=mesh,
        compiler_params=pltpu.CompilerParams(use_tc_tiling_on_sc=False,
                                             needs_layout_passes=False),
        out_type=jax.ShapeDtypeStruct((n_points, N_LEVELS * F_PER_LEVEL),
                                      jnp.float32),
        scratch_types=[
            pltpu.VMEM((C,), jnp.float32),
            pltpu.VMEM((C,), jnp.float32),
            pltpu.VMEM((C,), jnp.float32),
            pltpu.VMEM((16, C), jnp.int32),
            pltpu.VMEM((8, C), jnp.float32),
            pltpu.VMEM((16, C), jnp.float32),
            pltpu.VMEM((C, N_LEVELS * F_PER_LEVEL), jnp.float32),
            pltpu.SemaphoreType.DMA,
        ],
    )
    def enc_kernel(xa, xb, xc, tflat, out, xva, xvb, xvc,
                   idxv, wv, rowsv, encv, sem):
        wid = lax.axis_index("s") * NC + lax.axis_index("c")
        base0 = wid * pts_per_w
        xvs = (xva, xvb, xvc)

        def chunk_body(ch, carry):
            base = base0 + ch * C
            pltpu.sync_copy(xa.at[pl.ds(base, C)], xva)
            pltpu.sync_copy(xb.at[pl.ds(base, C)], xvb)
            pltpu.sync_copy(xc.at[pl.ds(base, C)], xvc)

            for l, (scale, res, dense) in enumerate(LEVELS):
                lvl_base = l * (T * F_PER_LEVEL)

                def pass_a(g, c2, scale=scale, res=res, dense=dense,
                           lvl_base=lvl_base):
                    off = g * L
                    sl = pl.ds(off, L)
                    coords = []
                    for d in range(3):
                        x01 = (xvs[d][sl] + 1.0) * 0.5
                        pos = x01 * jnp.float32(scale) + 0.5
                        pg = pos.astype(jnp.int32)
                        fr = pos - pg.astype(jnp.float32)
                        coords.append((pg, fr))
                    (pgx, fx), (pgy, fy), (pgz, fz) = coords
                    wx = (1.0 - fx, fx)
                    wy = (1.0 - fy, fy)
                    wz = (1.0 - fz, fz)
                    wxy = {(bx, by): wx[bx] * wy[by]
                           for bx in (0, 1) for by in (0, 1)}
                    if dense:
                        r1 = jnp.int32(res - 1)
                        cx = (jnp.minimum(pgx, r1), jnp.minimum(pgx + 1, r1))
                        cy0 = jnp.minimum(pgy, r1) * jnp.int32(res)
                        cy1 = jnp.minimum(pgy + 1, r1) * jnp.int32(res)
                        cz0 = jnp.minimum(pgz, r1) * jnp.int32(res * res)
                        cz1 = jnp.minimum(pgz + 1, r1) * jnp.int32(res * res)
                        cy = (cy0, cy1)
                        cz = (cz0, cz1)
                        for corner in range(8):
                            bx, by, bz = _corner_bits(corner)
                            t = cx[bx] + cy[by] + cz[bz]
                            a0 = (t + (t & jnp.int32(-128))
                                  + jnp.int32(lvl_base))
                            idxv[2 * corner, sl] = a0
                            idxv[2 * corner + 1, sl] = a0 + jnp.int32(128)
                            wv[corner, sl] = wxy[(bx, by)] * wz[bz]
                    else:
                        hx = (pgx, pgx + 1)
                        hy0 = pgy * jnp.int32(P1)
                        hy = (hy0, hy0 + jnp.int32(P1))
                        hz0 = pgz * jnp.int32(P2)
                        hz = (hz0, hz0 + jnp.int32(P2))
                        for corner in range(8):
                            bx, by, bz = _corner_bits(corner)
                            t = (hx[bx] ^ hy[by] ^ hz[bz]) & jnp.int32(T - 1)
                            a0 = (t + (t & jnp.int32(-128))
                                  + jnp.int32(lvl_base))
                            idxv[2 * corner, sl] = a0
                            idxv[2 * corner + 1, sl] = a0 + jnp.int32(128)
                            wv[corner, sl] = wxy[(bx, by)] * wz[bz]
                    return c2

                lax.fori_loop(0, groups, pass_a, 0)

                cps = [pltpu.async_copy(tflat.at[idxv.at[k]],
                                        rowsv.at[k], sem)
                       for k in range(16)]
                for cp in cps:
                    cp.wait()

                def pass_b(g, c2, l=l):
                    off = g * L
                    sl = pl.ds(off, L)
                    pvec = lax.iota(jnp.int32, L) + off
                    acc0 = jnp.zeros((L,), jnp.float32)
                    acc1 = jnp.zeros((L,), jnp.float32)
                    for corner in range(8):
                        w = wv[corner, sl]
                        f0 = rowsv[2 * corner, sl]
                        f1 = rowsv[2 * corner + 1, sl]
                        acc0 = acc0 + w * f0
                        acc1 = acc1 + w * f1
                    col0 = jnp.full((L,), 2 * l, jnp.int32)
                    plsc.store_scatter(encv, [pvec, col0], acc0)
                    plsc.store_scatter(encv, [pvec, col0 + 1], acc1)
                    return c2

                lax.fori_loop(0, groups, pass_b, 0)

            pltpu.sync_copy(encv, out.at[pl.ds(base, C)])
            return carry

        lax.fori_loop(0, n_chunks, chunk_body, 0)

    return enc_kernel


def _mlp_body(enc_ref, w1_ref, b1_ref, w2_ref, b2_ref, out_ref):
    h = jnp.dot(enc_ref[...], w1_ref[...],
                preferred_element_type=jnp.float32) + b1_ref[...]
    h = jnp.maximum(h, 0.0)
    out_ref[...] = jnp.dot(h, w2_ref[...],
                           preferred_element_type=jnp.float32) + b2_ref[...]


def _mlp(enc, W1, b1, W2, b2):
    n = enc.shape[0]
    bm = 4096
    return pl.pallas_call(
        _mlp_body,
        grid=(n // bm,),
        in_specs=[
            pl.BlockSpec((bm, N_LEVELS * F_PER_LEVEL), lambda i: (i, 0)),
            pl.BlockSpec((N_LEVELS * F_PER_LEVEL, 64), lambda i: (0, 0)),
            pl.BlockSpec((1, 64), lambda i: (0, 0)),
            pl.BlockSpec((64, 16), lambda i: (0, 0)),
            pl.BlockSpec((1, 16), lambda i: (0, 0)),
        ],
        out_specs=pl.BlockSpec((bm, 16), lambda i: (i, 0)),
        out_shape=jax.ShapeDtypeStruct((n, 16), jnp.float32),
    )(enc, W1, b1.reshape(1, 64), W2, b2.reshape(1, 16))


def kernel(x, table, W1, b1, W2, b2):
    n = x.shape[0]
    xT = x.T
    tflat = (table.reshape(N_LEVELS, T // 128, 128, F_PER_LEVEL)
             .transpose(0, 1, 3, 2).reshape(-1))
    enc = _make_enc_kernel(n)(xT[0], xT[1], xT[2], tflat)
    out = _mlp(enc, W1, b1, W2, b2)
    return out.reshape(x.shape[:-1] + (16,))


# speedup vs baseline: 4.0320x; 1.2229x over previous
import functools

import jax
import jax.numpy as jnp
import numpy as np
from jax import lax
from jax.experimental import pallas as pl
from jax.experimental.pallas import tpu as pltpu
from jax.experimental.pallas import tpu_sc as plsc

N_LEVELS = 16
F_PER_LEVEL = 2
T = 2 ** 19
BASE_RES = 16
PER_LEVEL_SCALE = 1.5
P1 = int(np.uint32(2654435761).view(np.int32))
P2 = int(np.uint32(805459861).view(np.int32))

NC, NS = 2, 16
NW = NC * NS
L = 16
C = 128


def _level_params():
    params = []
    for l in range(N_LEVELS):
        scale = BASE_RES * (PER_LEVEL_SCALE ** l) - 1.0
        res = int(np.ceil(scale)) + 1
        dense = (res ** 3) <= T
        params.append((float(scale), res, dense))
    return params

LEVELS = _level_params()


def _corner_bits(corner):
    return (corner >> 0) & 1, (corner >> 1) & 1, (corner >> 2) & 1


def _make_enc_kernel(n_points):
    pts_per_w = n_points // NW
    n_chunks = pts_per_w // C
    groups = C // L

    mesh = plsc.VectorSubcoreMesh(
        core_axis_name="c", subcore_axis_name="s",
        num_cores=NC, num_subcores=NS)

    @functools.partial(
        pl.kernel,
        mesh=mesh,
        compiler_params=pltpu.CompilerParams(use_tc_tiling_on_sc=False,
                                             needs_layout_passes=False),
        out_type=jax.ShapeDtypeStruct((n_points, N_LEVELS * F_PER_LEVEL),
                                      jnp.float32),
        scratch_types=[
            pltpu.VMEM((C,), jnp.float32),
            pltpu.VMEM((C,), jnp.float32),
            pltpu.VMEM((C,), jnp.float32),
            pltpu.VMEM((16 * C,), jnp.int32),
            pltpu.VMEM((16 * C,), jnp.int32),
            pltpu.VMEM((8, C), jnp.float32),
            pltpu.VMEM((8, C), jnp.float32),
            pltpu.VMEM((16 * C,), jnp.float32),
            pltpu.VMEM((16 * C,), jnp.float32),
            pltpu.VMEM((C, N_LEVELS * F_PER_LEVEL), jnp.float32),
            pltpu.SemaphoreType.DMA,
            pltpu.SemaphoreType.DMA,
        ],
    )
    def enc_kernel(xa, xb, xc, tflat, out, xva, xvb, xvc,
                   idx_a, idx_b, w_a, w_b, rows_a, rows_b, encv,
                   sem_a, sem_b):
        wid = lax.axis_index("s") * NC + lax.axis_index("c")
        base0 = wid * pts_per_w
        xvs = (xva, xvb, xvc)
        bufs = ((idx_a, w_a, rows_a, sem_a), (idx_b, w_b, rows_b, sem_b))

        def pass_a(l, idxv, wv):
            scale, res, dense = LEVELS[l]
            lvl_base = l * (T * F_PER_LEVEL)

            def body(g, c2):
                off = g * L
                sl = pl.ds(off, L)
                coords = []
                for d in range(3):
                    x01 = (xvs[d][sl] + 1.0) * 0.5
                    pos = x01 * jnp.float32(scale) + 0.5
                    pg = pos.astype(jnp.int32)
                    fr = pos - pg.astype(jnp.float32)
                    coords.append((pg, fr))
                (pgx, fx), (pgy, fy), (pgz, fz) = coords
                wx = (1.0 - fx, fx)
                wy = (1.0 - fy, fy)
                wz = (1.0 - fz, fz)
                wxy = {(bx, by): wx[bx] * wy[by]
                       for bx in (0, 1) for by in (0, 1)}
                if dense:
                    r1 = jnp.int32(res - 1)
                    cx = (jnp.minimum(pgx, r1), jnp.minimum(pgx + 1, r1))
                    cy = (jnp.minimum(pgy, r1) * jnp.int32(res),
                          jnp.minimum(pgy + 1, r1) * jnp.int32(res))
                    cz = (jnp.minimum(pgz, r1) * jnp.int32(res * res),
                          jnp.minimum(pgz + 1, r1) * jnp.int32(res * res))
                    for corner in range(8):
                        bx, by, bz = _corner_bits(corner)
                        t = cx[bx] + cy[by] + cz[bz]
                        a0 = t + (t & jnp.int32(-128)) + jnp.int32(lvl_base)
                        idxv[pl.ds(2 * corner * C + off, L)] = a0
                        idxv[pl.ds((2 * corner + 1) * C + off, L)] = (
                            a0 + jnp.int32(128))
                        wv[corner, sl] = wxy[(bx, by)] * wz[bz]
                else:
                    hx = (pgx, pgx + 1)
                    hy0 = pgy * jnp.int32(P1)
                    hy = (hy0, hy0 + jnp.int32(P1))
                    hz0 = pgz * jnp.int32(P2)
                    hz = (hz0, hz0 + jnp.int32(P2))
                    for corner in range(8):
                        bx, by, bz = _corner_bits(corner)
                        t = (hx[bx] ^ hy[by] ^ hz[bz]) & jnp.int32(T - 1)
                        a0 = t + (t & jnp.int32(-128)) + jnp.int32(lvl_base)
                        idxv[pl.ds(2 * corner * C + off, L)] = a0
                        idxv[pl.ds((2 * corner + 1) * C + off, L)] = (
                            a0 + jnp.int32(128))
                        wv[corner, sl] = wxy[(bx, by)] * wz[bz]
                return c2

            lax.fori_loop(0, groups, body, 0)

        def pass_b(l, wv, rowsv):
            def body(g, c2):
                off = g * L
                sl = pl.ds(off, L)
                pvec = lax.iota(jnp.int32, L) + off
                acc0 = jnp.zeros((L,), jnp.float32)
                acc1 = jnp.zeros((L,), jnp.float32)
                for corner in range(8):
                    w = wv[corner, sl]
                    f0 = rowsv[pl.ds(2 * corner * C + off, L)]
                    f1 = rowsv[pl.ds((2 * corner + 1) * C + off, L)]
                    acc0 = acc0 + w * f0
                    acc1 = acc1 + w * f1
                col0 = jnp.full((L,), 2 * l, jnp.int32)
                plsc.store_scatter(encv, [pvec, col0], acc0)
                plsc.store_scatter(encv, [pvec, col0 + 1], acc1)
                return c2

            lax.fori_loop(0, groups, body, 0)

        def chunk_body(ch, carry):
            base = base0 + ch * C
            pltpu.sync_copy(xa.at[pl.ds(base, C)], xva)
            pltpu.sync_copy(xb.at[pl.ds(base, C)], xvb)
            pltpu.sync_copy(xc.at[pl.ds(base, C)], xvc)

            cps = [None, None]
            pass_a(0, bufs[0][0], bufs[0][1])
            cps[0] = pltpu.async_copy(tflat.at[bufs[0][0]], bufs[0][2],
                                      bufs[0][3])
            for l in range(N_LEVELS):
                p = l & 1
                if l + 1 < N_LEVELS:
                    q = 1 - p
                    pass_a(l + 1, bufs[q][0], bufs[q][1])
                    cps[q] = pltpu.async_copy(tflat.at[bufs[q][0]],
                                              bufs[q][2], bufs[q][3])
                cps[p].wait()
                pass_b(l, bufs[p][1], bufs[p][2])

            pltpu.sync_copy(encv, out.at[pl.ds(base, C)])
            return carry

        lax.fori_loop(0, n_chunks, chunk_body, 0)

    return enc_kernel


def _mlp_body(enc_ref, w1_ref, b1_ref, w2_ref, b2_ref, out_ref):
    h = jnp.dot(enc_ref[...], w1_ref[...],
                preferred_element_type=jnp.float32) + b1_ref[...]
    h = jnp.maximum(h, 0.0)
    out_ref[...] = jnp.dot(h, w2_ref[...],
                           preferred_element_type=jnp.float32) + b2_ref[...]


def _mlp(enc, W1, b1, W2, b2):
    n = enc.shape[0]
    bm = 4096
    return pl.pallas_call(
        _mlp_body,
        grid=(n // bm,),
        in_specs=[
            pl.BlockSpec((bm, N_LEVELS * F_PER_LEVEL), lambda i: (i, 0)),
            pl.BlockSpec((N_LEVELS * F_PER_LEVEL, 64), lambda i: (0, 0)),
            pl.BlockSpec((1, 64), lambda i: (0, 0)),
            pl.BlockSpec((64, 16), lambda i: (0, 0)),
            pl.BlockSpec((1, 16), lambda i: (0, 0)),
        ],
        out_specs=pl.BlockSpec((bm, 16), lambda i: (i, 0)),
        out_shape=jax.ShapeDtypeStruct((n, 16), jnp.float32),
    )(enc, W1, b1.reshape(1, 64), W2, b2.reshape(1, 16))


def kernel(x, table, W1, b1, W2, b2):
    n = x.shape[0]
    xT = x.T
    tflat = (table.reshape(N_LEVELS, T // 128, 128, F_PER_LEVEL)
             .transpose(0, 1, 3, 2).reshape(-1))
    enc = _make_enc_kernel(n)(xT[0], xT[1], xT[2], tflat)
    out = _mlp(enc, W1, b1, W2, b2)
    return out.reshape(x.shape[:-1] + (16,))


# speedup vs baseline: 6.5285x; 1.6192x over previous
import functools

import jax
import jax.numpy as jnp
import numpy as np
from jax import lax
from jax.experimental import pallas as pl
from jax.experimental.pallas import tpu as pltpu
from jax.experimental.pallas import tpu_sc as plsc

N_LEVELS = 16
F_PER_LEVEL = 2
T = 2 ** 19
BASE_RES = 16
PER_LEVEL_SCALE = 1.5
P1 = int(np.uint32(2654435761).view(np.int32))
P2 = int(np.uint32(805459861).view(np.int32))

NC, NS = 2, 16
NW = NC * NS
L = 16
C = 128


def _level_params():
    params = []
    for l in range(N_LEVELS):
        scale = BASE_RES * (PER_LEVEL_SCALE ** l) - 1.0
        res = int(np.ceil(scale)) + 1
        dense = (res ** 3) <= T
        params.append((float(scale), res, dense))
    return params

LEVELS = _level_params()


def _corner_bits(corner):
    return (corner >> 0) & 1, (corner >> 1) & 1, (corner >> 2) & 1


def _make_repack_kernel():
    total = N_LEVELS * T * F_PER_LEVEL
    per_w = total // NW
    SLAB = 8192
    n_slabs = per_w // SLAB
    n_groups = SLAB // L

    mesh = plsc.VectorSubcoreMesh(
        core_axis_name="c", subcore_axis_name="s",
        num_cores=NC, num_subcores=NS)

    @functools.partial(
        pl.kernel,
        mesh=mesh,
        compiler_params=pltpu.CompilerParams(use_tc_tiling_on_sc=False,
                                             needs_layout_passes=False),
        out_type=jax.ShapeDtypeStruct((total,), jnp.float32),
        scratch_types=[
            pltpu.VMEM((SLAB,), jnp.float32),
            pltpu.VMEM((SLAB,), jnp.float32),
        ],
    )
    def repack_kernel(tflat, out, inb, outb):
        wid = lax.axis_index("s") * NC + lax.axis_index("c")
        base0 = wid * per_w
        iota = lax.iota(jnp.int32, L)
        lane_src = (lax.shift_right_logical(iota, jnp.int32(1))
                    + lax.shift_left(iota & jnp.int32(1), jnp.int32(7)))

        def slab_body(s, carry):
            base = base0 + s * SLAB
            pltpu.sync_copy(tflat.at[pl.ds(base, SLAB)], inb)

            def grp(g, c2):
                soff = (lax.shift_left(g, jnp.int32(4))
                        - lax.shift_left(g & jnp.int32(15), jnp.int32(3)))
                vals = plsc.load_gather(inb, [lane_src + soff])
                outb[pl.ds(g * L, L)] = vals
                return c2

            lax.fori_loop(0, n_groups, grp, 0)
            pltpu.sync_copy(outb, out.at[pl.ds(base, SLAB)])
            return carry

        lax.fori_loop(0, n_slabs, slab_body, 0)

    return repack_kernel


def _make_enc_kernel(n_points):
    pts_per_w = n_points // NW
    n_chunks = pts_per_w // C
    groups = C // L

    mesh = plsc.VectorSubcoreMesh(
        core_axis_name="c", subcore_axis_name="s",
        num_cores=NC, num_subcores=NS)

    @functools.partial(
        pl.kernel,
        mesh=mesh,
        compiler_params=pltpu.CompilerParams(use_tc_tiling_on_sc=False,
                                             needs_layout_passes=False),
        out_type=jax.ShapeDtypeStruct((n_points, N_LEVELS * F_PER_LEVEL),
                                      jnp.float32),
        scratch_types=[
            pltpu.VMEM((C,), jnp.float32),
            pltpu.VMEM((C,), jnp.float32),
            pltpu.VMEM((C,), jnp.float32),
            pltpu.VMEM((8 * C,), jnp.int32),
            pltpu.VMEM((8 * C,), jnp.int32),
            pltpu.VMEM((8, C), jnp.float32),
            pltpu.VMEM((8, C), jnp.float32),
            pltpu.VMEM((8, C), jnp.int32),
            pltpu.VMEM((8, C), jnp.int32),
            pltpu.VMEM((8 * C, 8), jnp.float32),
            pltpu.VMEM((8 * C, 8), jnp.float32),
            pltpu.VMEM((C, N_LEVELS * F_PER_LEVEL), jnp.float32),
            pltpu.SemaphoreType.DMA,
            pltpu.SemaphoreType.DMA,
        ],
    )
    def enc_kernel(xa, xb, xc, dense8, out, xva, xvb, xvc,
                   idx_a, idx_b, w_a, w_b, off_a, off_b, rows_a, rows_b,
                   encv, sem_a, sem_b):
        wid = lax.axis_index("s") * NC + lax.axis_index("c")
        base0 = wid * pts_per_w
        xvs = (xva, xvb, xvc)
        bufs = ((idx_a, w_a, off_a, rows_a, sem_a),
                (idx_b, w_b, off_b, rows_b, sem_b))

        def pass_a(l, idxv, wv, offv):
            scale, res, dense = LEVELS[l]
            lvl_q = l * (T // 4)

            def body(g, c2):
                off = g * L
                sl = pl.ds(off, L)
                coords = []
                for d in range(3):
                    x01 = (xvs[d][sl] + 1.0) * 0.5
                    pos = x01 * jnp.float32(scale) + 0.5
                    pg = pos.astype(jnp.int32)
                    fr = pos - pg.astype(jnp.float32)
                    coords.append((pg, fr))
                (pgx, fx), (pgy, fy), (pgz, fz) = coords
                wx = (1.0 - fx, fx)
                wy = (1.0 - fy, fy)
                wz = (1.0 - fz, fz)
                wxy = {(bx, by): wx[bx] * wy[by]
                       for bx in (0, 1) for by in (0, 1)}
                if dense:
                    r1 = jnp.int32(res - 1)
                    cx = (jnp.minimum(pgx, r1), jnp.minimum(pgx + 1, r1))
                    cy = (jnp.minimum(pgy, r1) * jnp.int32(res),
                          jnp.minimum(pgy + 1, r1) * jnp.int32(res))
                    cz = (jnp.minimum(pgz, r1) * jnp.int32(res * res),
                          jnp.minimum(pgz + 1, r1) * jnp.int32(res * res))
                    for corner in range(8):
                        bx, by, bz = _corner_bits(corner)
                        t = cx[bx] + cy[by] + cz[bz]
                        idxv[pl.ds(corner * C + off, L)] = (
                            lax.shift_right_logical(t, jnp.int32(2))
                            + jnp.int32(lvl_q))
                        offv[corner, sl] = lax.shift_left(
                            t & jnp.int32(3), jnp.int32(1))
                        wv[corner, sl] = wxy[(bx, by)] * wz[bz]
                else:
                    hx = (pgx, pgx + 1)
                    hy0 = pgy * jnp.int32(P1)
                    hy = (hy0, hy0 + jnp.int32(P1))
                    hz0 = pgz * jnp.int32(P2)
                    hz = (hz0, hz0 + jnp.int32(P2))
                    for corner in range(8):
                        bx, by, bz = _corner_bits(corner)
                        t = (hx[bx] ^ hy[by] ^ hz[bz]) & jnp.int32(T - 1)
                        idxv[pl.ds(corner * C + off, L)] = (
                            lax.shift_right_logical(t, jnp.int32(2))
                            + jnp.int32(lvl_q))
                        offv[corner, sl] = lax.shift_left(
                            t & jnp.int32(3), jnp.int32(1))
                        wv[corner, sl] = wxy[(bx, by)] * wz[bz]
                return c2

            lax.fori_loop(0, groups, body, 0)

        def pass_b(l, wv, offv, rowsv):
            def body(g, c2):
                off = g * L
                sl = pl.ds(off, L)
                pvec = lax.iota(jnp.int32, L) + off
                acc0 = jnp.zeros((L,), jnp.float32)
                acc1 = jnp.zeros((L,), jnp.float32)
                for corner in range(8):
                    w = wv[corner, sl]
                    rvec = pvec + jnp.int32(corner * C)
                    ov = offv[corner, sl]
                    f0 = plsc.load_gather(rowsv, [rvec, ov])
                    f1 = plsc.load_gather(rowsv, [rvec, ov + 1])
                    acc0 = acc0 + w * f0
                    acc1 = acc1 + w * f1
                col0 = jnp.full((L,), 2 * l, jnp.int32)
                plsc.store_scatter(encv, [pvec, col0], acc0)
                plsc.store_scatter(encv, [pvec, col0 + 1], acc1)
                return c2

            lax.fori_loop(0, groups, body, 0)

        def chunk_body(ch, carry):
            base = base0 + ch * C
            pltpu.sync_copy(xa.at[pl.ds(base, C)], xva)
            pltpu.sync_copy(xb.at[pl.ds(base, C)], xvb)
            pltpu.sync_copy(xc.at[pl.ds(base, C)], xvc)

            cps = [None, None]
            pass_a(0, bufs[0][0], bufs[0][1], bufs[0][2])
            cps[0] = pltpu.async_copy(dense8.at[bufs[0][0]], bufs[0][3],
                                      bufs[0][4])
            for l in range(N_LEVELS):
                p = l & 1
                if l + 1 < N_LEVELS:
                    q = 1 - p
                    pass_a(l + 1, bufs[q][0], bufs[q][1], bufs[q][2])
                    cps[q] = pltpu.async_copy(dense8.at[bufs[q][0]],
                                              bufs[q][3], bufs[q][4])
                cps[p].wait()
                pass_b(l, bufs[p][1], bufs[p][2], bufs[p][3])

            pltpu.sync_copy(encv, out.at[pl.ds(base, C)])
            return carry

        lax.fori_loop(0, n_chunks, chunk_body, 0)

    return enc_kernel


def _mlp_body(enc_ref, w1_ref, b1_ref, w2_ref, b2_ref, out_ref):
    h = jnp.dot(enc_ref[...], w1_ref[...],
                preferred_element_type=jnp.float32) + b1_ref[...]
    h = jnp.maximum(h, 0.0)
    out_ref[...] = jnp.dot(h, w2_ref[...],
                           preferred_element_type=jnp.float32) + b2_ref[...]


def _mlp(enc, W1, b1, W2, b2):
    n = enc.shape[0]
    bm = 4096
    return pl.pallas_call(
        _mlp_body,
        grid=(n // bm,),
        in_specs=[
            pl.BlockSpec((bm, N_LEVELS * F_PER_LEVEL), lambda i: (i, 0)),
            pl.BlockSpec((N_LEVELS * F_PER_LEVEL, 64), lambda i: (0, 0)),
            pl.BlockSpec((1, 64), lambda i: (0, 0)),
            pl.BlockSpec((64, 16), lambda i: (0, 0)),
            pl.BlockSpec((1, 16), lambda i: (0, 0)),
        ],
        out_specs=pl.BlockSpec((bm, 16), lambda i: (i, 0)),
        out_shape=jax.ShapeDtypeStruct((n, 16), jnp.float32),
    )(enc, W1, b1.reshape(1, 64), W2, b2.reshape(1, 16))


def kernel(x, table, W1, b1, W2, b2):
    n = x.shape[0]
    xT = x.T
    tflat = (table.reshape(N_LEVELS, T // 128, 128, F_PER_LEVEL)
             .transpose(0, 1, 3, 2).reshape(-1))
    dense8 = _make_repack_kernel()(tflat).reshape(
        N_LEVELS * T * F_PER_LEVEL // 8, 8)
    enc = _make_enc_kernel(n)(xT[0], xT[1], xT[2], dense8)
    out = _mlp(enc, W1, b1, W2, b2)
    return out.reshape(x.shape[:-1] + (16,))


# speedup vs baseline: 7.5826x; 1.1615x over previous
import functools

import jax
import jax.numpy as jnp
import numpy as np
from jax import lax
from jax.experimental import pallas as pl
from jax.experimental.pallas import tpu as pltpu
from jax.experimental.pallas import tpu_sc as plsc

N_LEVELS = 16
F_PER_LEVEL = 2
T = 2 ** 19
BASE_RES = 16
PER_LEVEL_SCALE = 1.5
P1 = int(np.uint32(2654435761).view(np.int32))
P2 = int(np.uint32(805459861).view(np.int32))

NC, NS = 2, 16
NW = NC * NS
L = 16
C = 128


def _level_params():
    params = []
    for l in range(N_LEVELS):
        scale = BASE_RES * (PER_LEVEL_SCALE ** l) - 1.0
        res = int(np.ceil(scale)) + 1
        dense = (res ** 3) <= T
        params.append((float(scale), res, dense))
    return params

LEVELS = _level_params()
N_LOCAL = 2
LOC_ROWS = tuple(LEVELS[l][1] ** 3 // 4 for l in range(N_LOCAL))


def _corner_bits(corner):
    return (corner >> 0) & 1, (corner >> 1) & 1, (corner >> 2) & 1


def _make_repack_kernel():
    total = N_LEVELS * T * F_PER_LEVEL
    per_w = total // NW
    SLAB = 16384
    n_slabs = per_w // SLAB
    n_groups = SLAB // L

    mesh = plsc.VectorSubcoreMesh(
        core_axis_name="c", subcore_axis_name="s",
        num_cores=NC, num_subcores=NS)

    @functools.partial(
        pl.kernel,
        mesh=mesh,
        compiler_params=pltpu.CompilerParams(use_tc_tiling_on_sc=False,
                                             needs_layout_passes=False),
        out_type=jax.ShapeDtypeStruct((total,), jnp.float32),
        scratch_types=[
            pltpu.VMEM((SLAB,), jnp.float32),
            pltpu.VMEM((SLAB,), jnp.float32),
        ],
    )
    def repack_kernel(tflat, out, inb, outb):
        wid = lax.axis_index("s") * NC + lax.axis_index("c")
        base0 = wid * per_w
        iota = lax.iota(jnp.int32, L)
        lane_src = (lax.shift_right_logical(iota, jnp.int32(1))
                    + lax.shift_left(iota & jnp.int32(1), jnp.int32(7)))

        def slab_body(s, carry):
            base = base0 + s * SLAB
            pltpu.sync_copy(tflat.at[pl.ds(base, SLAB)], inb)

            def grp(g, c2):
                soff = (lax.shift_left(g, jnp.int32(4))
                        - lax.shift_left(g & jnp.int32(15), jnp.int32(3)))
                vals = plsc.load_gather(inb, [lane_src + soff])
                outb[pl.ds(g * L, L)] = vals
                return c2

            lax.fori_loop(0, n_groups, grp, 0)
            pltpu.sync_copy(outb, out.at[pl.ds(base, SLAB)])
            return carry

        lax.fori_loop(0, n_slabs, slab_body, 0)

    return repack_kernel


def _make_enc_kernel(n_points):
    pts_per_w = n_points // NW
    n_chunks = pts_per_w // C
    groups = C // L

    mesh = plsc.VectorSubcoreMesh(
        core_axis_name="c", subcore_axis_name="s",
        num_cores=NC, num_subcores=NS)

    @functools.partial(
        pl.kernel,
        mesh=mesh,
        compiler_params=pltpu.CompilerParams(use_tc_tiling_on_sc=False,
                                             needs_layout_passes=False),
        out_type=jax.ShapeDtypeStruct((n_points, N_LEVELS * F_PER_LEVEL),
                                      jnp.float32),
        scratch_types=[
            pltpu.VMEM((C,), jnp.float32),
            pltpu.VMEM((C,), jnp.float32),
            pltpu.VMEM((C,), jnp.float32),
            pltpu.VMEM((8 * C,), jnp.int32),
            pltpu.VMEM((8 * C,), jnp.int32),
            pltpu.VMEM((8, C), jnp.float32),
            pltpu.VMEM((8, C), jnp.float32),
            pltpu.VMEM((8, C), jnp.int32),
            pltpu.VMEM((8, C), jnp.int32),
            pltpu.VMEM((8 * C, 8), jnp.float32),
            pltpu.VMEM((8 * C, 8), jnp.float32),
            pltpu.VMEM((C, N_LEVELS * F_PER_LEVEL), jnp.float32),
            pltpu.VMEM((LOC_ROWS[0], 8), jnp.float32),
            pltpu.VMEM((LOC_ROWS[1], 8), jnp.float32),
            pltpu.SemaphoreType.DMA,
            pltpu.SemaphoreType.DMA,
        ],
    )
    def enc_kernel(xa, xb, xc, dense8, out, xva, xvb, xvc,
                   idx_a, idx_b, w_a, w_b, off_a, off_b, rows_a, rows_b,
                   encv, loc0, loc1, sem_a, sem_b):
        wid = lax.axis_index("s") * NC + lax.axis_index("c")
        base0 = wid * pts_per_w
        xvs = (xva, xvb, xvc)
        bufs = ((idx_a, w_a, off_a, rows_a, sem_a),
                (idx_b, w_b, off_b, rows_b, sem_b))

        def pass_a(l, idxv, wv, offv, local=False):
            scale, res, dense = LEVELS[l]
            lvl_q = 0 if local else l * (T // 4)

            def body(g, c2):
                off = g * L
                sl = pl.ds(off, L)
                coords = []
                for d in range(3):
                    x01 = (xvs[d][sl] + 1.0) * 0.5
                    pos = x01 * jnp.float32(scale) + 0.5
                    pg = pos.astype(jnp.int32)
                    fr = pos - pg.astype(jnp.float32)
                    coords.append((pg, fr))
                (pgx, fx), (pgy, fy), (pgz, fz) = coords
                wx = (1.0 - fx, fx)
                wy = (1.0 - fy, fy)
                wz = (1.0 - fz, fz)
                wxy = {(bx, by): wx[bx] * wy[by]
                       for bx in (0, 1) for by in (0, 1)}
                if dense:
                    r1 = jnp.int32(res - 1)
                    cx = (jnp.minimum(pgx, r1), jnp.minimum(pgx + 1, r1))
                    cy = (jnp.minimum(pgy, r1) * jnp.int32(res),
                          jnp.minimum(pgy + 1, r1) * jnp.int32(res))
                    cz = (jnp.minimum(pgz, r1) * jnp.int32(res * res),
                          jnp.minimum(pgz + 1, r1) * jnp.int32(res * res))
                    for corner in range(8):
                        bx, by, bz = _corner_bits(corner)
                        t = cx[bx] + cy[by] + cz[bz]
                        idxv[pl.ds(corner * C + off, L)] = (
                            lax.shift_right_logical(t, jnp.int32(2))
                            + jnp.int32(lvl_q))
                        offv[corner, sl] = lax.shift_left(
                            t & jnp.int32(3), jnp.int32(1))
                        wv[corner, sl] = wxy[(bx, by)] * wz[bz]
                else:
                    hx = (pgx, pgx + 1)
                    hy0 = pgy * jnp.int32(P1)
                    hy = (hy0, hy0 + jnp.int32(P1))
                    hz0 = pgz * jnp.int32(P2)
                    hz = (hz0, hz0 + jnp.int32(P2))
                    for corner in range(8):
                        bx, by, bz = _corner_bits(corner)
                        t = (hx[bx] ^ hy[by] ^ hz[bz]) & jnp.int32(T - 1)
                        idxv[pl.ds(corner * C + off, L)] = (
                            lax.shift_right_logical(t, jnp.int32(2))
                            + jnp.int32(lvl_q))
                        offv[corner, sl] = lax.shift_left(
                            t & jnp.int32(3), jnp.int32(1))
                        wv[corner, sl] = wxy[(bx, by)] * wz[bz]
                return c2

            lax.fori_loop(0, groups, body, 0)

        def pass_b(l, wv, offv, rowsv):
            def body(g, c2):
                off = g * L
                sl = pl.ds(off, L)
                pvec = lax.iota(jnp.int32, L) + off
                acc0 = jnp.zeros((L,), jnp.float32)
                acc1 = jnp.zeros((L,), jnp.float32)
                for corner in range(8):
                    w = wv[corner, sl]
                    rvec = pvec + jnp.int32(corner * C)
                    ov = offv[corner, sl]
                    f0 = plsc.load_gather(rowsv, [rvec, ov])
                    f1 = plsc.load_gather(rowsv, [rvec, ov + 1])
                    acc0 = acc0 + w * f0
                    acc1 = acc1 + w * f1
                col0 = jnp.full((L,), 2 * l, jnp.int32)
                plsc.store_scatter(encv, [pvec, col0], acc0)
                plsc.store_scatter(encv, [pvec, col0 + 1], acc1)
                return c2

            lax.fori_loop(0, groups, body, 0)

        def pass_b_local(l, wv, offv, idxv, loc):
            def body(g, c2):
                off = g * L
                sl = pl.ds(off, L)
                pvec = lax.iota(jnp.int32, L) + off
                acc0 = jnp.zeros((L,), jnp.float32)
                acc1 = jnp.zeros((L,), jnp.float32)
                for corner in range(8):
                    w = wv[corner, sl]
                    rvec = idxv[pl.ds(corner * C + off, L)]
                    ov = offv[corner, sl]
                    f0 = plsc.load_gather(loc, [rvec, ov])
                    f1 = plsc.load_gather(loc, [rvec, ov + 1])
                    acc0 = acc0 + w * f0
                    acc1 = acc1 + w * f1
                col0 = jnp.full((L,), 2 * l, jnp.int32)
                plsc.store_scatter(encv, [pvec, col0], acc0)
                plsc.store_scatter(encv, [pvec, col0 + 1], acc1)
                return c2

            lax.fori_loop(0, groups, body, 0)

        pltpu.sync_copy(dense8.at[pl.ds(0, LOC_ROWS[0])], loc0)
        pltpu.sync_copy(dense8.at[pl.ds(T // 4, LOC_ROWS[1])], loc1)
        locs = (loc0, loc1)

        def chunk_body(ch, carry):
            base = base0 + ch * C
            pltpu.sync_copy(xa.at[pl.ds(base, C)], xva)
            pltpu.sync_copy(xb.at[pl.ds(base, C)], xvb)
            pltpu.sync_copy(xc.at[pl.ds(base, C)], xvc)

            cps = [None, None]
            pass_a(N_LOCAL, bufs[0][0], bufs[0][1], bufs[0][2])
            cps[0] = pltpu.async_copy(dense8.at[bufs[0][0]], bufs[0][3],
                                      bufs[0][4])
            for l in range(N_LOCAL):
                pass_a(l, bufs[1][0], bufs[1][1], bufs[1][2], local=True)
                pass_b_local(l, bufs[1][1], bufs[1][2], bufs[1][0], locs[l])
            for i, l in enumerate(range(N_LOCAL, N_LEVELS)):
                p = i & 1
                if l + 1 < N_LEVELS:
                    q = 1 - p
                    pass_a(l + 1, bufs[q][0], bufs[q][1], bufs[q][2])
                    cps[q] = pltpu.async_copy(dense8.at[bufs[q][0]],
                                              bufs[q][3], bufs[q][4])
                cps[p].wait()
                pass_b(l, bufs[p][1], bufs[p][2], bufs[p][3])

            pltpu.sync_copy(encv, out.at[pl.ds(base, C)])
            return carry

        lax.fori_loop(0, n_chunks, chunk_body, 0)

    return enc_kernel


def _mlp_body(enc_ref, w1_ref, b1_ref, w2_ref, b2_ref, out_ref):
    h = jnp.dot(enc_ref[...], w1_ref[...],
                preferred_element_type=jnp.float32) + b1_ref[...]
    h = jnp.maximum(h, 0.0)
    out_ref[...] = jnp.dot(h, w2_ref[...],
                           preferred_element_type=jnp.float32) + b2_ref[...]


def _mlp(enc, W1, b1, W2, b2):
    n = enc.shape[0]
    bm = 4096
    return pl.pallas_call(
        _mlp_body,
        grid=(n // bm,),
        in_specs=[
            pl.BlockSpec((bm, N_LEVELS * F_PER_LEVEL), lambda i: (i, 0)),
            pl.BlockSpec((N_LEVELS * F_PER_LEVEL, 64), lambda i: (0, 0)),
            pl.BlockSpec((1, 64), lambda i: (0, 0)),
            pl.BlockSpec((64, 16), lambda i: (0, 0)),
            pl.BlockSpec((1, 16), lambda i: (0, 0)),
        ],
        out_specs=pl.BlockSpec((bm, 16), lambda i: (i, 0)),
        out_shape=jax.ShapeDtypeStruct((n, 16), jnp.float32),
    )(enc, W1, b1.reshape(1, 64), W2, b2.reshape(1, 16))


def kernel(x, table, W1, b1, W2, b2):
    n = x.shape[0]
    xT = x.T
    tflat = (table.reshape(N_LEVELS, T // 128, 128, F_PER_LEVEL)
             .transpose(0, 1, 3, 2).reshape(-1))
    dense8 = _make_repack_kernel()(tflat).reshape(
        N_LEVELS * T * F_PER_LEVEL // 8, 8)
    enc = _make_enc_kernel(n)(xT[0], xT[1], xT[2], dense8)
    out = _mlp(enc, W1, b1, W2, b2)
    return out.reshape(x.shape[:-1] + (16,))


# speedup vs baseline: 8.2791x; 1.0919x over previous
import functools

import jax
import jax.numpy as jnp
import numpy as np
from jax import lax
from jax.experimental import pallas as pl
from jax.experimental.pallas import tpu as pltpu
from jax.experimental.pallas import tpu_sc as plsc

N_LEVELS = 16
F_PER_LEVEL = 2
T = 2 ** 19
BASE_RES = 16
PER_LEVEL_SCALE = 1.5
P1 = int(np.uint32(2654435761).view(np.int32))
P2 = int(np.uint32(805459861).view(np.int32))

NC, NS = 2, 16
NW = NC * NS
L = 16
C = 256


def _level_params():
    params = []
    for l in range(N_LEVELS):
        scale = BASE_RES * (PER_LEVEL_SCALE ** l) - 1.0
        res = int(np.ceil(scale)) + 1
        dense = (res ** 3) <= T
        params.append((float(scale), res, dense))
    return params

LEVELS = _level_params()
N_LOCAL = 2
LOC_ROWS = tuple(LEVELS[l][1] ** 3 // 4 for l in range(N_LOCAL))


def _corner_bits(corner):
    return (corner >> 0) & 1, (corner >> 1) & 1, (corner >> 2) & 1


def _make_repack_kernel():
    total = N_LEVELS * T * F_PER_LEVEL
    per_w = total // NW
    SLAB = 16384
    n_slabs = per_w // SLAB
    n_groups = SLAB // L

    mesh = plsc.VectorSubcoreMesh(
        core_axis_name="c", subcore_axis_name="s",
        num_cores=NC, num_subcores=NS)

    @functools.partial(
        pl.kernel,
        mesh=mesh,
        compiler_params=pltpu.CompilerParams(use_tc_tiling_on_sc=False,
                                             needs_layout_passes=False),
        out_type=jax.ShapeDtypeStruct((total,), jnp.float32),
        scratch_types=[
            pltpu.VMEM((SLAB,), jnp.float32),
            pltpu.VMEM((SLAB,), jnp.float32),
        ],
    )
    def repack_kernel(tflat, out, inb, outb):
        wid = lax.axis_index("s") * NC + lax.axis_index("c")
        base0 = wid * per_w
        iota = lax.iota(jnp.int32, L)
        lane_src = (lax.shift_right_logical(iota, jnp.int32(1))
                    + lax.shift_left(iota & jnp.int32(1), jnp.int32(7)))

        def slab_body(s, carry):
            base = base0 + s * SLAB
            pltpu.sync_copy(tflat.at[pl.ds(base, SLAB)], inb)

            def grp(g, c2):
                soff = (lax.shift_left(g, jnp.int32(4))
                        - lax.shift_left(g & jnp.int32(15), jnp.int32(3)))
                vals = plsc.load_gather(inb, [lane_src + soff])
                outb[pl.ds(g * L, L)] = vals
                return c2

            lax.fori_loop(0, n_groups, grp, 0)
            pltpu.sync_copy(outb, out.at[pl.ds(base, SLAB)])
            return carry

        lax.fori_loop(0, n_slabs, slab_body, 0)

    return repack_kernel


def _make_enc_kernel(n_points):
    pts_per_w = n_points // NW
    n_chunks = pts_per_w // C
    groups = C // L

    mesh = plsc.VectorSubcoreMesh(
        core_axis_name="c", subcore_axis_name="s",
        num_cores=NC, num_subcores=NS)

    @functools.partial(
        pl.kernel,
        mesh=mesh,
        compiler_params=pltpu.CompilerParams(use_tc_tiling_on_sc=False,
                                             needs_layout_passes=False),
        out_type=jax.ShapeDtypeStruct((n_points, N_LEVELS * F_PER_LEVEL),
                                      jnp.float32),
        scratch_types=[
            pltpu.VMEM((C,), jnp.float32),
            pltpu.VMEM((C,), jnp.float32),
            pltpu.VMEM((C,), jnp.float32),
            pltpu.VMEM((8 * C,), jnp.int32),
            pltpu.VMEM((8 * C,), jnp.int32),
            pltpu.VMEM((8, C), jnp.float32),
            pltpu.VMEM((8, C), jnp.float32),
            pltpu.VMEM((8, C), jnp.int32),
            pltpu.VMEM((8, C), jnp.int32),
            pltpu.VMEM((8 * C, 8), jnp.float32),
            pltpu.VMEM((8 * C, 8), jnp.float32),
            pltpu.VMEM((C, N_LEVELS * F_PER_LEVEL), jnp.float32),
            pltpu.VMEM((LOC_ROWS[0], 8), jnp.float32),
            pltpu.VMEM((LOC_ROWS[1], 8), jnp.float32),
            pltpu.SemaphoreType.DMA,
            pltpu.SemaphoreType.DMA,
        ],
    )
    def enc_kernel(xa, xb, xc, dense8, out, xva, xvb, xvc,
                   idx_a, idx_b, w_a, w_b, off_a, off_b, rows_a, rows_b,
                   encv, loc0, loc1, sem_a, sem_b):
        wid = lax.axis_index("s") * NC + lax.axis_index("c")
        base0 = wid * pts_per_w
        xvs = (xva, xvb, xvc)
        bufs = ((idx_a, w_a, off_a, rows_a, sem_a),
                (idx_b, w_b, off_b, rows_b, sem_b))

        def pass_a(l, idxv, wv, offv, local=False):
            scale, res, dense = LEVELS[l]
            lvl_q = 0 if local else l * (T // 4)

            def body(g, c2):
                off = g * L
                sl = pl.ds(off, L)
                coords = []
                for d in range(3):
                    x01 = (xvs[d][sl] + 1.0) * 0.5
                    pos = x01 * jnp.float32(scale) + 0.5
                    pg = pos.astype(jnp.int32)
                    fr = pos - pg.astype(jnp.float32)
                    coords.append((pg, fr))
                (pgx, fx), (pgy, fy), (pgz, fz) = coords
                wx = (1.0 - fx, fx)
                wy = (1.0 - fy, fy)
                wz = (1.0 - fz, fz)
                wxy = {(bx, by): wx[bx] * wy[by]
                       for bx in (0, 1) for by in (0, 1)}
                if dense:
                    r1 = jnp.int32(res - 1)
                    cx = (jnp.minimum(pgx, r1), jnp.minimum(pgx + 1, r1))
                    cy = (jnp.minimum(pgy, r1) * jnp.int32(res),
                          jnp.minimum(pgy + 1, r1) * jnp.int32(res))
                    cz = (jnp.minimum(pgz, r1) * jnp.int32(res * res),
                          jnp.minimum(pgz + 1, r1) * jnp.int32(res * res))
                    for corner in range(8):
                        bx, by, bz = _corner_bits(corner)
                        t = cx[bx] + cy[by] + cz[bz]
                        idxv[pl.ds(corner * C + off, L)] = (
                            lax.shift_right_logical(t, jnp.int32(2))
                            + jnp.int32(lvl_q))
                        offv[corner, sl] = lax.shift_left(
                            t & jnp.int32(3), jnp.int32(1))
                        wv[corner, sl] = wxy[(bx, by)] * wz[bz]
                else:
                    hx = (pgx, pgx + 1)
                    hy0 = pgy * jnp.int32(P1)
                    hy = (hy0, hy0 + jnp.int32(P1))
                    hz0 = pgz * jnp.int32(P2)
                    hz = (hz0, hz0 + jnp.int32(P2))
                    for corner in range(8):
                        bx, by, bz = _corner_bits(corner)
                        t = (hx[bx] ^ hy[by] ^ hz[bz]) & jnp.int32(T - 1)
                        idxv[pl.ds(corner * C + off, L)] = (
                            lax.shift_right_logical(t, jnp.int32(2))
                            + jnp.int32(lvl_q))
                        offv[corner, sl] = lax.shift_left(
                            t & jnp.int32(3), jnp.int32(1))
                        wv[corner, sl] = wxy[(bx, by)] * wz[bz]
                return c2

            lax.fori_loop(0, groups, body, 0)

        def pass_b(l, wv, offv, rowsv):
            def body(g, c2):
                off = g * L
                sl = pl.ds(off, L)
                pvec = lax.iota(jnp.int32, L) + off
                acc0 = jnp.zeros((L,), jnp.float32)
                acc1 = jnp.zeros((L,), jnp.float32)
                for corner in range(8):
                    w = wv[corner, sl]
                    rvec = pvec + jnp.int32(corner * C)
                    ov = offv[corner, sl]
                    f0 = plsc.load_gather(rowsv, [rvec, ov])
                    f1 = plsc.load_gather(rowsv, [rvec, ov + 1])
                    acc0 = acc0 + w * f0
                    acc1 = acc1 + w * f1
                col0 = jnp.full((L,), 2 * l, jnp.int32)
                plsc.store_scatter(encv, [pvec, col0], acc0)
                plsc.store_scatter(encv, [pvec, col0 + 1], acc1)
                return c2

            lax.fori_loop(0, groups, body, 0)

        def pass_b_local(l, wv, offv, idxv, loc):
            def body(g, c2):
                off = g * L
                sl = pl.ds(off, L)
                pvec = lax.iota(jnp.int32, L) + off
                acc0 = jnp.zeros((L,), jnp.float32)
                acc1 = jnp.zeros((L,), jnp.float32)
                for corner in range(8):
                    w = wv[corner, sl]
                    rvec = idxv[pl.ds(corner * C + off, L)]
                    ov = offv[corner, sl]
                    f0 = plsc.load_gather(loc, [rvec, ov])
                    f1 = plsc.load_gather(loc, [rvec, ov + 1])
                    acc0 = acc0 + w * f0
                    acc1 = acc1 + w * f1
                col0 = jnp.full((L,), 2 * l, jnp.int32)
                plsc.store_scatter(encv, [pvec, col0], acc0)
                plsc.store_scatter(encv, [pvec, col0 + 1], acc1)
                return c2

            lax.fori_loop(0, groups, body, 0)

        pltpu.sync_copy(dense8.at[pl.ds(0, LOC_ROWS[0])], loc0)
        pltpu.sync_copy(dense8.at[pl.ds(T // 4, LOC_ROWS[1])], loc1)
        locs = (loc0, loc1)

        def chunk_body(ch, carry):
            base = base0 + ch * C
            pltpu.sync_copy(xa.at[pl.ds(base, C)], xva)
            pltpu.sync_copy(xb.at[pl.ds(base, C)], xvb)
            pltpu.sync_copy(xc.at[pl.ds(base, C)], xvc)

            cps = [None, None]
            pass_a(N_LOCAL, bufs[0][0], bufs[0][1], bufs[0][2])
            cps[0] = pltpu.async_copy(dense8.at[bufs[0][0]], bufs[0][3],
                                      bufs[0][4])
            for l in range(N_LOCAL):
                pass_a(l, bufs[1][0], bufs[1][1], bufs[1][2], local=True)
                pass_b_local(l, bufs[1][1], bufs[1][2], bufs[1][0], locs[l])
            for i, l in enumerate(range(N_LOCAL, N_LEVELS)):
                p = i & 1
                if l + 1 < N_LEVELS:
                    q = 1 - p
                    pass_a(l + 1, bufs[q][0], bufs[q][1], bufs[q][2])
                    cps[q] = pltpu.async_copy(dense8.at[bufs[q][0]],
                                              bufs[q][3], bufs[q][4])
                cps[p].wait()
                pass_b(l, bufs[p][1], bufs[p][2], bufs[p][3])

            pltpu.sync_copy(encv, out.at[pl.ds(base, C)])
            return carry

        lax.fori_loop(0, n_chunks, chunk_body, 0)

    return enc_kernel


def _mlp_body(enc_ref, w1_ref, b1_ref, w2_ref, b2_ref, out_ref):
    h = jnp.dot(enc_ref[...], w1_ref[...],
                preferred_element_type=jnp.float32) + b1_ref[...]
    h = jnp.maximum(h, 0.0)
    out_ref[...] = jnp.dot(h, w2_ref[...],
                           preferred_element_type=jnp.float32) + b2_ref[...]


def _mlp(enc, W1, b1, W2, b2):
    n = enc.shape[0]
    bm = 4096
    return pl.pallas_call(
        _mlp_body,
        grid=(n // bm,),
        in_specs=[
            pl.BlockSpec((bm, N_LEVELS * F_PER_LEVEL), lambda i: (i, 0)),
            pl.BlockSpec((N_LEVELS * F_PER_LEVEL, 64), lambda i: (0, 0)),
            pl.BlockSpec((1, 64), lambda i: (0, 0)),
            pl.BlockSpec((64, 16), lambda i: (0, 0)),
            pl.BlockSpec((1, 16), lambda i: (0, 0)),
        ],
        out_specs=pl.BlockSpec((bm, 16), lambda i: (i, 0)),
        out_shape=jax.ShapeDtypeStruct((n, 16), jnp.float32),
    )(enc, W1, b1.reshape(1, 64), W2, b2.reshape(1, 16))


def kernel(x, table, W1, b1, W2, b2):
    n = x.shape[0]
    xT = x.T
    tflat = (table.reshape(N_LEVELS, T // 128, 128, F_PER_LEVEL)
             .transpose(0, 1, 3, 2).reshape(-1))
    dense8 = _make_repack_kernel()(tflat).reshape(
        N_LEVELS * T * F_PER_LEVEL // 8, 8)
    enc = _make_enc_kernel(n)(xT[0], xT[1], xT[2], dense8)
    out = _mlp(enc, W1, b1, W2, b2)
    return out.reshape(x.shape[:-1] + (16,))


# speedup vs baseline: 8.7223x; 1.0535x over previous
import functools

import jax
import jax.numpy as jnp
import numpy as np
from jax import lax
from jax.experimental import pallas as pl
from jax.experimental.pallas import tpu as pltpu
from jax.experimental.pallas import tpu_sc as plsc

N_LEVELS = 16
F_PER_LEVEL = 2
T = 2 ** 19
BASE_RES = 16
PER_LEVEL_SCALE = 1.5
P1 = int(np.uint32(2654435761).view(np.int32))
P2 = int(np.uint32(805459861).view(np.int32))

NC, NS = 2, 16
NW = NC * NS
L = 16
C = 256


def _level_params():
    params = []
    for l in range(N_LEVELS):
        scale = BASE_RES * (PER_LEVEL_SCALE ** l) - 1.0
        res = int(np.ceil(scale)) + 1
        dense = (res ** 3) <= T
        params.append((float(scale), res, dense))
    return params

LEVELS = _level_params()
N_LOCAL = 2
LOC_ROWS = tuple(LEVELS[l][1] ** 3 // 4 for l in range(N_LOCAL))
SH_LEVELS = (2, 3)
SH_ROWS = tuple(LEVELS[l][1] ** 3 // 4 for l in SH_LEVELS)
SH_BASE = {l: sum(SH_ROWS[:i]) for i, l in enumerate(SH_LEVELS)}


def _corner_bits(corner):
    return (corner >> 0) & 1, (corner >> 1) & 1, (corner >> 2) & 1


def _make_repack_kernel():
    total = N_LEVELS * T * F_PER_LEVEL
    per_w = total // NW
    SLAB = 16384
    n_slabs = per_w // SLAB
    n_groups = SLAB // L

    mesh = plsc.VectorSubcoreMesh(
        core_axis_name="c", subcore_axis_name="s",
        num_cores=NC, num_subcores=NS)

    @functools.partial(
        pl.kernel,
        mesh=mesh,
        compiler_params=pltpu.CompilerParams(use_tc_tiling_on_sc=False,
                                             needs_layout_passes=False),
        out_type=jax.ShapeDtypeStruct((total,), jnp.float32),
        scratch_types=[
            pltpu.VMEM((SLAB,), jnp.float32),
            pltpu.VMEM((SLAB,), jnp.float32),
        ],
    )
    def repack_kernel(tflat, out, inb, outb):
        wid = lax.axis_index("s") * NC + lax.axis_index("c")
        base0 = wid * per_w
        iota = lax.iota(jnp.int32, L)
        lane_src = (lax.shift_right_logical(iota, jnp.int32(1))
                    + lax.shift_left(iota & jnp.int32(1), jnp.int32(7)))

        def slab_body(s, carry):
            base = base0 + s * SLAB
            pltpu.sync_copy(tflat.at[pl.ds(base, SLAB)], inb)

            def grp(g, c2):
                soff = (lax.shift_left(g, jnp.int32(4))
                        - lax.shift_left(g & jnp.int32(15), jnp.int32(3)))
                vals = plsc.load_gather(inb, [lane_src + soff])
                outb[pl.ds(g * L, L)] = vals
                return c2

            lax.fori_loop(0, n_groups, grp, 0)
            pltpu.sync_copy(outb, out.at[pl.ds(base, SLAB)])
            return carry

        lax.fori_loop(0, n_slabs, slab_body, 0)

    return repack_kernel


def _make_enc_kernel(n_points):
    pts_per_w = n_points // NW
    n_chunks = pts_per_w // C
    groups = C // L

    mesh = plsc.VectorSubcoreMesh(
        core_axis_name="c", subcore_axis_name="s",
        num_cores=NC, num_subcores=NS)

    @functools.partial(
        pl.kernel,
        mesh=mesh,
        compiler_params=pltpu.CompilerParams(use_tc_tiling_on_sc=False,
                                             needs_layout_passes=False),
        out_type=jax.ShapeDtypeStruct((n_points, N_LEVELS * F_PER_LEVEL),
                                      jnp.float32),
        scratch_types=[
            pltpu.VMEM((C,), jnp.float32),
            pltpu.VMEM((C,), jnp.float32),
            pltpu.VMEM((C,), jnp.float32),
            pltpu.VMEM((8 * C,), jnp.int32),
            pltpu.VMEM((8 * C,), jnp.int32),
            pltpu.VMEM((8, C), jnp.float32),
            pltpu.VMEM((8, C), jnp.float32),
            pltpu.VMEM((8, C), jnp.int32),
            pltpu.VMEM((8, C), jnp.int32),
            pltpu.VMEM((8 * C, 8), jnp.float32),
            pltpu.VMEM((8 * C, 8), jnp.float32),
            pltpu.VMEM((C, N_LEVELS * F_PER_LEVEL), jnp.float32),
            pltpu.VMEM((LOC_ROWS[0], 8), jnp.float32),
            pltpu.VMEM((LOC_ROWS[1], 8), jnp.float32),
            pltpu.VMEM_SHARED((sum(SH_ROWS), 8), jnp.float32),
            pltpu.SemaphoreType.DMA,
            pltpu.SemaphoreType.DMA,
        ],
    )
    def enc_kernel(xa, xb, xc, dense8, out, xva, xvb, xvc,
                   idx_a, idx_b, w_a, w_b, off_a, off_b, rows_a, rows_b,
                   encv, loc0, loc1, shared23, sem_a, sem_b):
        wid = lax.axis_index("s") * NC + lax.axis_index("c")
        base0 = wid * pts_per_w
        xvs = (xva, xvb, xvc)
        bufs = ((idx_a, w_a, off_a, rows_a, sem_a),
                (idx_b, w_b, off_b, rows_b, sem_b))

        def pass_a(l, idxv, wv, offv, local=False):
            scale, res, dense = LEVELS[l]
            if local:
                lvl_q = 0
            elif l in SH_BASE:
                lvl_q = SH_BASE[l]
            else:
                lvl_q = l * (T // 4)

            def body(g, c2):
                off = g * L
                sl = pl.ds(off, L)
                coords = []
                for d in range(3):
                    x01 = (xvs[d][sl] + 1.0) * 0.5
                    pos = x01 * jnp.float32(scale) + 0.5
                    pg = pos.astype(jnp.int32)
                    fr = pos - pg.astype(jnp.float32)
                    coords.append((pg, fr))
                (pgx, fx), (pgy, fy), (pgz, fz) = coords
                wx = (1.0 - fx, fx)
                wy = (1.0 - fy, fy)
                wz = (1.0 - fz, fz)
                wxy = {(bx, by): wx[bx] * wy[by]
                       for bx in (0, 1) for by in (0, 1)}
                if dense:
                    r1 = jnp.int32(res - 1)
                    cx = (jnp.minimum(pgx, r1), jnp.minimum(pgx + 1, r1))
                    cy = (jnp.minimum(pgy, r1) * jnp.int32(res),
                          jnp.minimum(pgy + 1, r1) * jnp.int32(res))
                    cz = (jnp.minimum(pgz, r1) * jnp.int32(res * res),
                          jnp.minimum(pgz + 1, r1) * jnp.int32(res * res))
                    for corner in range(8):
                        bx, by, bz = _corner_bits(corner)
                        t = cx[bx] + cy[by] + cz[bz]
                        idxv[pl.ds(corner * C + off, L)] = (
                            lax.shift_right_logical(t, jnp.int32(2))
                            + jnp.int32(lvl_q))
                        offv[corner, sl] = lax.shift_left(
                            t & jnp.int32(3), jnp.int32(1))
                        wv[corner, sl] = wxy[(bx, by)] * wz[bz]
                else:
                    hx = (pgx, pgx + 1)
                    hy0 = pgy * jnp.int32(P1)
                    hy = (hy0, hy0 + jnp.int32(P1))
                    hz0 = pgz * jnp.int32(P2)
                    hz = (hz0, hz0 + jnp.int32(P2))
                    for corner in range(8):
                        bx, by, bz = _corner_bits(corner)
                        t = (hx[bx] ^ hy[by] ^ hz[bz]) & jnp.int32(T - 1)
                        idxv[pl.ds(corner * C + off, L)] = (
                            lax.shift_right_logical(t, jnp.int32(2))
                            + jnp.int32(lvl_q))
                        offv[corner, sl] = lax.shift_left(
                            t & jnp.int32(3), jnp.int32(1))
                        wv[corner, sl] = wxy[(bx, by)] * wz[bz]
                return c2

            lax.fori_loop(0, groups, body, 0)

        def pass_b(l, wv, offv, rowsv):
            def body(g, c2):
                off = g * L
                sl = pl.ds(off, L)
                pvec = lax.iota(jnp.int32, L) + off
                acc0 = jnp.zeros((L,), jnp.float32)
                acc1 = jnp.zeros((L,), jnp.float32)
                for corner in range(8):
                    w = wv[corner, sl]
                    rvec = pvec + jnp.int32(corner * C)
                    ov = offv[corner, sl]
                    f0 = plsc.load_gather(rowsv, [rvec, ov])
                    f1 = plsc.load_gather(rowsv, [rvec, ov + 1])
                    acc0 = acc0 + w * f0
                    acc1 = acc1 + w * f1
                col0 = jnp.full((L,), 2 * l, jnp.int32)
                plsc.store_scatter(encv, [pvec, col0], acc0)
                plsc.store_scatter(encv, [pvec, col0 + 1], acc1)
                return c2

            lax.fori_loop(0, groups, body, 0)

        def pass_b_local(l, wv, offv, idxv, loc):
            def body(g, c2):
                off = g * L
                sl = pl.ds(off, L)
                pvec = lax.iota(jnp.int32, L) + off
                acc0 = jnp.zeros((L,), jnp.float32)
                acc1 = jnp.zeros((L,), jnp.float32)
                for corner in range(8):
                    w = wv[corner, sl]
                    rvec = idxv[pl.ds(corner * C + off, L)]
                    ov = offv[corner, sl]
                    f0 = plsc.load_gather(loc, [rvec, ov])
                    f1 = plsc.load_gather(loc, [rvec, ov + 1])
                    acc0 = acc0 + w * f0
                    acc1 = acc1 + w * f1
                col0 = jnp.full((L,), 2 * l, jnp.int32)
                plsc.store_scatter(encv, [pvec, col0], acc0)
                plsc.store_scatter(encv, [pvec, col0 + 1], acc1)
                return c2

            lax.fori_loop(0, groups, body, 0)

        pltpu.sync_copy(dense8.at[pl.ds(0, LOC_ROWS[0])], loc0)
        pltpu.sync_copy(dense8.at[pl.ds(T // 4, LOC_ROWS[1])], loc1)
        locs = (loc0, loc1)
        @pl.when(lax.axis_index("s") == 0)
        def _stage_shared():
            for i, l in enumerate(SH_LEVELS):
                pltpu.sync_copy(
                    dense8.at[pl.ds(l * (T // 4), SH_ROWS[i])],
                    shared23.at[pl.ds(SH_BASE[l], SH_ROWS[i])])
        plsc.subcore_barrier()

        def fire(l, buf):
            src = shared23 if l in SH_BASE else dense8
            return pltpu.async_copy(src.at[buf[0]], buf[3], buf[4])

        def chunk_body(ch, carry):
            base = base0 + ch * C
            pltpu.sync_copy(xa.at[pl.ds(base, C)], xva)
            pltpu.sync_copy(xb.at[pl.ds(base, C)], xvb)
            pltpu.sync_copy(xc.at[pl.ds(base, C)], xvc)

            cps = [None, None]
            pass_a(N_LOCAL, bufs[0][0], bufs[0][1], bufs[0][2])
            cps[0] = fire(N_LOCAL, bufs[0])
            for l in range(N_LOCAL):
                pass_a(l, bufs[1][0], bufs[1][1], bufs[1][2], local=True)
                pass_b_local(l, bufs[1][1], bufs[1][2], bufs[1][0], locs[l])
            for i, l in enumerate(range(N_LOCAL, N_LEVELS)):
                p = i & 1
                if l + 1 < N_LEVELS:
                    q = 1 - p
                    pass_a(l + 1, bufs[q][0], bufs[q][1], bufs[q][2])
                    cps[q] = fire(l + 1, bufs[q])
                cps[p].wait()
                pass_b(l, bufs[p][1], bufs[p][2], bufs[p][3])

            pltpu.sync_copy(encv, out.at[pl.ds(base, C)])
            return carry

        lax.fori_loop(0, n_chunks, chunk_body, 0)

    return enc_kernel


def _mlp_body(enc_ref, w1_ref, b1_ref, w2_ref, b2_ref, out_ref):
    h = jnp.dot(enc_ref[...], w1_ref[...],
                preferred_element_type=jnp.float32) + b1_ref[...]
    h = jnp.maximum(h, 0.0)
    out_ref[...] = jnp.dot(h, w2_ref[...],
                           preferred_element_type=jnp.float32) + b2_ref[...]


def _mlp(enc, W1, b1, W2, b2):
    n = enc.shape[0]
    bm = 4096
    return pl.pallas_call(
        _mlp_body,
        grid=(n // bm,),
        in_specs=[
            pl.BlockSpec((bm, N_LEVELS * F_PER_LEVEL), lambda i: (i, 0)),
            pl.BlockSpec((N_LEVELS * F_PER_LEVEL, 64), lambda i: (0, 0)),
            pl.BlockSpec((1, 64), lambda i: (0, 0)),
            pl.BlockSpec((64, 16), lambda i: (0, 0)),
            pl.BlockSpec((1, 16), lambda i: (0, 0)),
        ],
        out_specs=pl.BlockSpec((bm, 16), lambda i: (i, 0)),
        out_shape=jax.ShapeDtypeStruct((n, 16), jnp.float32),
    )(enc, W1, b1.reshape(1, 64), W2, b2.reshape(1, 16))


def kernel(x, table, W1, b1, W2, b2):
    n = x.shape[0]
    xT = x.T
    tflat = (table.reshape(N_LEVELS, T // 128, 128, F_PER_LEVEL)
             .transpose(0, 1, 3, 2).reshape(-1))
    dense8 = _make_repack_kernel()(tflat).reshape(
        N_LEVELS * T * F_PER_LEVEL // 8, 8)
    enc = _make_enc_kernel(n)(xT[0], xT[1], xT[2], dense8)
    out = _mlp(enc, W1, b1, W2, b2)
    return out.reshape(x.shape[:-1] + (16,))


# speedup vs baseline: 8.9986x; 1.0317x over previous
import functools

import jax
import jax.numpy as jnp
import numpy as np
from jax import lax
from jax.experimental import pallas as pl
from jax.experimental.pallas import tpu as pltpu
from jax.experimental.pallas import tpu_sc as plsc

N_LEVELS = 16
F_PER_LEVEL = 2
T = 2 ** 19
BASE_RES = 16
PER_LEVEL_SCALE = 1.5
P1 = int(np.uint32(2654435761).view(np.int32))
P2 = int(np.uint32(805459861).view(np.int32))

NC, NS = 2, 16
NW = NC * NS
L = 16
C = 256


def _level_params():
    params = []
    for l in range(N_LEVELS):
        scale = BASE_RES * (PER_LEVEL_SCALE ** l) - 1.0
        res = int(np.ceil(scale)) + 1
        dense = (res ** 3) <= T
        params.append((float(scale), res, dense))
    return params

LEVELS = _level_params()
N_LOCAL = 2
LOC_ROWS = tuple(LEVELS[l][1] ** 3 // 4 for l in range(N_LOCAL))
SH_LEVELS = (2, 3)
SH_ROWS = tuple(LEVELS[l][1] ** 3 // 4 for l in SH_LEVELS)
SH_BASE = {l: sum(SH_ROWS[:i]) for i, l in enumerate(SH_LEVELS)}


def _corner_bits(corner):
    return (corner >> 0) & 1, (corner >> 1) & 1, (corner >> 2) & 1


def _make_repack_kernel():
    total = N_LEVELS * T * F_PER_LEVEL
    per_w = total // NW
    SLAB = 16384
    n_slabs = per_w // SLAB
    n_groups = SLAB // L

    mesh = plsc.VectorSubcoreMesh(
        core_axis_name="c", subcore_axis_name="s",
        num_cores=NC, num_subcores=NS)

    @functools.partial(
        pl.kernel,
        mesh=mesh,
        compiler_params=pltpu.CompilerParams(use_tc_tiling_on_sc=False,
                                             needs_layout_passes=False),
        out_type=jax.ShapeDtypeStruct((total,), jnp.float32),
        scratch_types=[
            pltpu.VMEM((SLAB,), jnp.float32),
            pltpu.VMEM((SLAB,), jnp.float32),
            pltpu.VMEM((SLAB,), jnp.float32),
            pltpu.SemaphoreType.DMA,
            pltpu.SemaphoreType.DMA,
        ],
    )
    def repack_kernel(tflat, out, in_a, in_b, outb, sem_a, sem_b):
        wid = lax.axis_index("s") * NC + lax.axis_index("c")
        base0 = wid * per_w
        iota = lax.iota(jnp.int32, L)
        lane_src = (lax.shift_right_logical(iota, jnp.int32(1))
                    + lax.shift_left(iota & jnp.int32(1), jnp.int32(7)))
        inbufs = ((in_a, sem_a), (in_b, sem_b))

        cps = [None, None]
        cps[0] = pltpu.async_copy(tflat.at[pl.ds(base0, SLAB)], in_a, sem_a)
        for s in range(n_slabs):
            p = s & 1
            if s + 1 < n_slabs:
                q = 1 - p
                cps[q] = pltpu.async_copy(
                    tflat.at[pl.ds(base0 + (s + 1) * SLAB, SLAB)],
                    inbufs[q][0], inbufs[q][1])
            cps[p].wait()
            inb = inbufs[p][0]

            def grp(g, c2, inb=inb):
                soff = (lax.shift_left(g, jnp.int32(4))
                        - lax.shift_left(g & jnp.int32(15), jnp.int32(3)))
                vals = plsc.load_gather(inb, [lane_src + soff])
                outb[pl.ds(g * L, L)] = vals
                return c2

            lax.fori_loop(0, n_groups, grp, 0)
            pltpu.sync_copy(outb, out.at[pl.ds(base0 + s * SLAB, SLAB)])

    return repack_kernel


def _make_enc_kernel(n_points):
    pts_per_w = n_points // NW
    n_chunks = pts_per_w // C
    groups = C // L

    mesh = plsc.VectorSubcoreMesh(
        core_axis_name="c", subcore_axis_name="s",
        num_cores=NC, num_subcores=NS)

    @functools.partial(
        pl.kernel,
        mesh=mesh,
        compiler_params=pltpu.CompilerParams(use_tc_tiling_on_sc=False,
                                             needs_layout_passes=False),
        out_type=jax.ShapeDtypeStruct((n_points, N_LEVELS * F_PER_LEVEL),
                                      jnp.float32),
        scratch_types=[
            pltpu.VMEM((C,), jnp.float32),
            pltpu.VMEM((C,), jnp.float32),
            pltpu.VMEM((C,), jnp.float32),
            pltpu.VMEM((8 * C,), jnp.int32),
            pltpu.VMEM((8 * C,), jnp.int32),
            pltpu.VMEM((8, C), jnp.float32),
            pltpu.VMEM((8, C), jnp.float32),
            pltpu.VMEM((8, C), jnp.int32),
            pltpu.VMEM((8, C), jnp.int32),
            pltpu.VMEM((8 * C, 8), jnp.float32),
            pltpu.VMEM((8 * C, 8), jnp.float32),
            pltpu.VMEM((C, N_LEVELS * F_PER_LEVEL), jnp.float32),
            pltpu.VMEM((LOC_ROWS[0], 8), jnp.float32),
            pltpu.VMEM((LOC_ROWS[1], 8), jnp.float32),
            pltpu.VMEM_SHARED((sum(SH_ROWS), 8), jnp.float32),
            pltpu.SemaphoreType.DMA,
            pltpu.SemaphoreType.DMA,
        ],
    )
    def enc_kernel(xa, xb, xc, dense8, out, xva, xvb, xvc,
                   idx_a, idx_b, w_a, w_b, off_a, off_b, rows_a, rows_b,
                   encv, loc0, loc1, shared23, sem_a, sem_b):
        wid = lax.axis_index("s") * NC + lax.axis_index("c")
        base0 = wid * pts_per_w
        xvs = (xva, xvb, xvc)
        bufs = ((idx_a, w_a, off_a, rows_a, sem_a),
                (idx_b, w_b, off_b, rows_b, sem_b))

        def pass_a(l, idxv, wv, offv, local=False):
            scale, res, dense = LEVELS[l]
            if local:
                lvl_q = 0
            elif l in SH_BASE:
                lvl_q = SH_BASE[l]
            else:
                lvl_q = l * (T // 4)

            def body(g, c2):
                off = g * L
                sl = pl.ds(off, L)
                coords = []
                for d in range(3):
                    x01 = (xvs[d][sl] + 1.0) * 0.5
                    pos = x01 * jnp.float32(scale) + 0.5
                    pg = pos.astype(jnp.int32)
                    fr = pos - pg.astype(jnp.float32)
                    coords.append((pg, fr))
                (pgx, fx), (pgy, fy), (pgz, fz) = coords
                wx = (1.0 - fx, fx)
                wy = (1.0 - fy, fy)
                wz = (1.0 - fz, fz)
                wxy = {(bx, by): wx[bx] * wy[by]
                       for bx in (0, 1) for by in (0, 1)}
                if dense:
                    r1 = jnp.int32(res - 1)
                    cx = (jnp.minimum(pgx, r1), jnp.minimum(pgx + 1, r1))
                    cy = (jnp.minimum(pgy, r1) * jnp.int32(res),
                          jnp.minimum(pgy + 1, r1) * jnp.int32(res))
                    cz = (jnp.minimum(pgz, r1) * jnp.int32(res * res),
                          jnp.minimum(pgz + 1, r1) * jnp.int32(res * res))
                    for corner in range(8):
                        bx, by, bz = _corner_bits(corner)
                        t = cx[bx] + cy[by] + cz[bz]
                        idxv[pl.ds(corner * C + off, L)] = (
                            lax.shift_right_logical(t, jnp.int32(2))
                            + jnp.int32(lvl_q))
                        offv[corner, sl] = lax.shift_left(
                            t & jnp.int32(3), jnp.int32(1))
                        wv[corner, sl] = wxy[(bx, by)] * wz[bz]
                else:
                    hx = (pgx, pgx + 1)
                    hy0 = pgy * jnp.int32(P1)
                    hy = (hy0, hy0 + jnp.int32(P1))
                    hz0 = pgz * jnp.int32(P2)
                    hz = (hz0, hz0 + jnp.int32(P2))
                    for corner in range(8):
                        bx, by, bz = _corner_bits(corner)
                        t = (hx[bx] ^ hy[by] ^ hz[bz]) & jnp.int32(T - 1)
                        idxv[pl.ds(corner * C + off, L)] = (
                            lax.shift_right_logical(t, jnp.int32(2))
                            + jnp.int32(lvl_q))
                        offv[corner, sl] = lax.shift_left(
                            t & jnp.int32(3), jnp.int32(1))
                        wv[corner, sl] = wxy[(bx, by)] * wz[bz]
                return c2

            lax.fori_loop(0, groups, body, 0)

        def pass_b(l, wv, offv, rowsv):
            def body(g, c2):
                off = g * L
                sl = pl.ds(off, L)
                pvec = lax.iota(jnp.int32, L) + off
                acc0 = jnp.zeros((L,), jnp.float32)
                acc1 = jnp.zeros((L,), jnp.float32)
                for corner in range(8):
                    w = wv[corner, sl]
                    rvec = pvec + jnp.int32(corner * C)
                    ov = offv[corner, sl]
                    f0 = plsc.load_gather(rowsv, [rvec, ov])
                    f1 = plsc.load_gather(rowsv, [rvec, ov + 1])
                    acc0 = acc0 + w * f0
                    acc1 = acc1 + w * f1
                col0 = jnp.full((L,), 2 * l, jnp.int32)
                plsc.store_scatter(encv, [pvec, col0], acc0)
                plsc.store_scatter(encv, [pvec, col0 + 1], acc1)
                return c2

            lax.fori_loop(0, groups, body, 0)

        def pass_b_local(l, wv, offv, idxv, loc):
            def body(g, c2):
                off = g * L
                sl = pl.ds(off, L)
                pvec = lax.iota(jnp.int32, L) + off
                acc0 = jnp.zeros((L,), jnp.float32)
                acc1 = jnp.zeros((L,), jnp.float32)
                for corner in range(8):
                    w = wv[corner, sl]
                    rvec = idxv[pl.ds(corner * C + off, L)]
                    ov = offv[corner, sl]
                    f0 = plsc.load_gather(loc, [rvec, ov])
                    f1 = plsc.load_gather(loc, [rvec, ov + 1])
                    acc0 = acc0 + w * f0
                    acc1 = acc1 + w * f1
                col0 = jnp.full((L,), 2 * l, jnp.int32)
                plsc.store_scatter(encv, [pvec, col0], acc0)
                plsc.store_scatter(encv, [pvec, col0 + 1], acc1)
                return c2

            lax.fori_loop(0, groups, body, 0)

        pltpu.sync_copy(dense8.at[pl.ds(0, LOC_ROWS[0])], loc0)
        pltpu.sync_copy(dense8.at[pl.ds(T // 4, LOC_ROWS[1])], loc1)
        locs = (loc0, loc1)
        @pl.when(lax.axis_index("s") == 0)
        def _stage_shared():
            for i, l in enumerate(SH_LEVELS):
                pltpu.sync_copy(
                    dense8.at[pl.ds(l * (T // 4), SH_ROWS[i])],
                    shared23.at[pl.ds(SH_BASE[l], SH_ROWS[i])])
        plsc.subcore_barrier()

        def fire(l, buf):
            src = shared23 if l in SH_BASE else dense8
            return pltpu.async_copy(src.at[buf[0]], buf[3], buf[4])

        def chunk_body(ch, carry):
            base = base0 + ch * C
            pltpu.sync_copy(xa.at[pl.ds(base, C)], xva)
            pltpu.sync_copy(xb.at[pl.ds(base, C)], xvb)
            pltpu.sync_copy(xc.at[pl.ds(base, C)], xvc)

            cps = [None, None]
            pass_a(N_LOCAL, bufs[0][0], bufs[0][1], bufs[0][2])
            cps[0] = fire(N_LOCAL, bufs[0])
            for l in range(N_LOCAL):
                pass_a(l, bufs[1][0], bufs[1][1], bufs[1][2], local=True)
                pass_b_local(l, bufs[1][1], bufs[1][2], bufs[1][0], locs[l])
            for i, l in enumerate(range(N_LOCAL, N_LEVELS)):
                p = i & 1
                if l + 1 < N_LEVELS:
                    q = 1 - p
                    pass_a(l + 1, bufs[q][0], bufs[q][1], bufs[q][2])
                    cps[q] = fire(l + 1, bufs[q])
                cps[p].wait()
                pass_b(l, bufs[p][1], bufs[p][2], bufs[p][3])

            pltpu.sync_copy(encv, out.at[pl.ds(base, C)])
            return carry

        lax.fori_loop(0, n_chunks, chunk_body, 0)

    return enc_kernel


def _mlp_body(enc_ref, w1_ref, b1_ref, w2_ref, b2_ref, out_ref):
    h = jnp.dot(enc_ref[...], w1_ref[...],
                preferred_element_type=jnp.float32) + b1_ref[...]
    h = jnp.maximum(h, 0.0)
    out_ref[...] = jnp.dot(h, w2_ref[...],
                           preferred_element_type=jnp.float32) + b2_ref[...]


def _mlp(enc, W1, b1, W2, b2):
    n = enc.shape[0]
    bm = 4096
    return pl.pallas_call(
        _mlp_body,
        grid=(n // bm,),
        in_specs=[
            pl.BlockSpec((bm, N_LEVELS * F_PER_LEVEL), lambda i: (i, 0)),
            pl.BlockSpec((N_LEVELS * F_PER_LEVEL, 64), lambda i: (0, 0)),
            pl.BlockSpec((1, 64), lambda i: (0, 0)),
            pl.BlockSpec((64, 16), lambda i: (0, 0)),
            pl.BlockSpec((1, 16), lambda i: (0, 0)),
        ],
        out_specs=pl.BlockSpec((bm, 16), lambda i: (i, 0)),
        out_shape=jax.ShapeDtypeStruct((n, 16), jnp.float32),
    )(enc, W1, b1.reshape(1, 64), W2, b2.reshape(1, 16))


def kernel(x, table, W1, b1, W2, b2):
    n = x.shape[0]
    xT = x.T
    tflat = (table.reshape(N_LEVELS, T // 128, 128, F_PER_LEVEL)
             .transpose(0, 1, 3, 2).reshape(-1))
    dense8 = _make_repack_kernel()(tflat).reshape(
        N_LEVELS * T * F_PER_LEVEL // 8, 8)
    enc = _make_enc_kernel(n)(xT[0], xT[1], xT[2], dense8)
    out = _mlp(enc, W1, b1, W2, b2)
    return out.reshape(x.shape[:-1] + (16,))
